# Initial kernel scaffold; baseline (speedup 1.0000x reference)
import jax
import jax.numpy as jnp
from jax.experimental import pallas as pl


def kernel(x, w_enc, w_dec, b_dec):
    raise NotImplementedError("write your pallas kernel here")


# speedup vs baseline: 19.0087x; 19.0087x over previous
import functools

import jax
import jax.numpy as jnp
from jax.experimental import pallas as pl
from jax.experimental.pallas import tpu as pltpu

K = 32
BLOCK_T = 512
N_BISECT = 26


def _body(x_ref, we_ref, wd_ref, b_ref, xhat_ref, xhid_ref):
    xb = x_ref[...] - b_ref[...]
    h = jax.lax.dot_general(
        xb, we_ref[...], (((1,), (1,)), ((), ())),
        preferred_element_type=jnp.float32)
    lo = jnp.min(h, axis=1, keepdims=True)
    hi = jnp.max(h, axis=1, keepdims=True)
    for _ in range(N_BISECT):
        mid = 0.5 * (lo + hi)
        cnt = jnp.sum(jnp.where(h >= mid, 1.0, 0.0), axis=1, keepdims=True)
        ge = cnt >= float(K)
        lo = jnp.where(ge, mid, lo)
        hi = jnp.where(ge, hi, mid)
    xhid = jnp.where(h >= lo, h, 0.0)
    xhid_ref[...] = xhid
    xhat_ref[...] = jax.lax.dot_general(
        xhid, wd_ref[...], (((1,), (1,)), ((), ())),
        preferred_element_type=jnp.float32) + b_ref[...]


@jax.jit
def kernel(x, w_enc, w_dec, b_dec):
    b, s, d_model = x.shape
    d_feat = w_enc.shape[0]
    n_tok = b * s
    xf = x.reshape(n_tok, d_model)
    b2 = b_dec.reshape(1, d_model)
    grid = (n_tok // BLOCK_T,)
    xhat, xhid = pl.pallas_call(
        _body,
        grid=grid,
        in_specs=[
            pl.BlockSpec((BLOCK_T, d_model), lambda i: (i, 0)),
            pl.BlockSpec((d_feat, d_model), lambda i: (0, 0)),
            pl.BlockSpec((d_model, d_feat), lambda i: (0, 0)),
            pl.BlockSpec((1, d_model), lambda i: (0, 0)),
        ],
        out_specs=[
            pl.BlockSpec((BLOCK_T, d_model), lambda i: (i, 0)),
            pl.BlockSpec((BLOCK_T, d_feat), lambda i: (i, 0)),
        ],
        out_shape=[
            jax.ShapeDtypeStruct((n_tok, d_model), jnp.float32),
            jax.ShapeDtypeStruct((n_tok, d_feat), jnp.float32),
        ],
        compiler_params=pltpu.CompilerParams(
            dimension_semantics=("arbitrary",),
        ),
    )(xf, w_enc, w_dec, b2)
    return (xhat.reshape(b, s, d_model), xhid.reshape(b, s, d_feat))


# speedup vs baseline: 22.6031x; 1.1891x over previous
import functools

import jax
import jax.numpy as jnp
from jax.experimental import pallas as pl
from jax.experimental.pallas import tpu as pltpu

K = 32
BLOCK_T = 512
N_BISECT = 20


def _body(x_ref, we_ref, wd_ref, b_ref, xhat_ref, xhid_ref):
    xb = x_ref[...] - b_ref[...]
    h = jax.lax.dot_general(
        xb, we_ref[...], (((1,), (1,)), ((), ())),
        preferred_element_type=jnp.float32)
    lo = jnp.min(h, axis=1, keepdims=True)
    hi = jnp.max(h, axis=1, keepdims=True)
    for _ in range(N_BISECT):
        mid = 0.5 * (lo + hi)
        cnt = jnp.sum(jnp.where(h >= mid, 1.0, 0.0), axis=1, keepdims=True)
        ge = cnt >= float(K)
        lo = jnp.where(ge, mid, lo)
        hi = jnp.where(ge, hi, mid)
    xhid = jnp.where(h >= lo, h, 0.0)
    xhid_ref[...] = xhid
    xhat_ref[...] = jax.lax.dot_general(
        xhid, wd_ref[...], (((1,), (1,)), ((), ())),
        preferred_element_type=jnp.float32) + b_ref[...]


@jax.jit
def kernel(x, w_enc, w_dec, b_dec):
    b, s, d_model = x.shape
    d_feat = w_enc.shape[0]
    n_tok = b * s
    xf = x.reshape(n_tok, d_model)
    b2 = b_dec.reshape(1, d_model)
    grid = (n_tok // BLOCK_T,)
    xhat, xhid = pl.pallas_call(
        _body,
        grid=grid,
        in_specs=[
            pl.BlockSpec((BLOCK_T, d_model), lambda i: (i, 0)),
            pl.BlockSpec((d_feat, d_model), lambda i: (0, 0)),
            pl.BlockSpec((d_model, d_feat), lambda i: (0, 0)),
            pl.BlockSpec((1, d_model), lambda i: (0, 0)),
        ],
        out_specs=[
            pl.BlockSpec((BLOCK_T, d_model), lambda i: (i, 0)),
            pl.BlockSpec((BLOCK_T, d_feat), lambda i: (i, 0)),
        ],
        out_shape=[
            jax.ShapeDtypeStruct((n_tok, d_model), jnp.float32),
            jax.ShapeDtypeStruct((n_tok, d_feat), jnp.float32),
        ],
        compiler_params=pltpu.CompilerParams(
            dimension_semantics=("arbitrary",),
        ),
    )(xf, w_enc, w_dec, b2)
    return (xhat.reshape(b, s, d_model), xhid.reshape(b, s, d_feat))
